# Initial kernel scaffold; baseline (speedup 1.0000x reference)
import jax
import jax.numpy as jnp
from jax.experimental import pallas as pl


def kernel(inputs, targets, table):
    raise NotImplementedError("write your pallas kernel here")


# speedup vs baseline: 1.3991x; 1.3991x over previous
import functools

import jax
import jax.numpy as jnp
from jax import lax
from jax.experimental import pallas as pl
from jax.experimental.pallas import tpu as pltpu
from jax.experimental.pallas import tpu_sc as plsc

V = 1000
VP = 1024
NC, NS = 2, 16
NW = NC * NS
N = 1024 * 50
PER_W = N // NW
C = 64
NCH = PER_W // C


def _lse_body(t_ref, o_ref):
    x = t_ref[...]
    m = jnp.max(x, axis=1, keepdims=True)
    s = jnp.sum(jnp.exp(x - m), axis=1, keepdims=True)
    o_ref[...] = m + jnp.log(s)


def _loss_body(p_ref, o_ref):
    s = jnp.sum(p_ref[...], axis=1, keepdims=True)
    o_ref[...] = jnp.sum(s, axis=0, keepdims=True) * (1.0 / N)


_mesh = plsc.VectorSubcoreMesh(core_axis_name="c", subcore_axis_name="s")


@functools.partial(
    pl.kernel,
    mesh=_mesh,
    compiler_params=pltpu.CompilerParams(
        use_tc_tiling_on_sc=False, needs_layout_passes=False
    ),
    out_type=[
        jax.ShapeDtypeStruct((N, V), jnp.float32),
        jax.ShapeDtypeStruct((NW * 32,), jnp.float32),
    ],
    scratch_types=[
        pltpu.VMEM((PER_W,), jnp.int32),
        pltpu.VMEM((PER_W,), jnp.int32),
        pltpu.VMEM((VP,), jnp.float32),
        pltpu.VMEM((C, V), jnp.float32),
        pltpu.VMEM((16,), jnp.float32),
        pltpu.SemaphoreType.DMA,
    ],
)
def _sc_gather(table_hbm, idx_hbm, tgt_hbm, lse_hbm, out_hbm, part_hbm,
               idx_v, tgt_v, lse_v, rows_v, tmp_v, sem):
    wid = lax.axis_index("s") * NC + lax.axis_index("c")
    base = wid * PER_W
    pltpu.sync_copy(idx_hbm.at[pl.ds(base, PER_W)], idx_v)
    pltpu.sync_copy(tgt_hbm.at[pl.ds(base, PER_W)], tgt_v)
    pltpu.sync_copy(lse_hbm, lse_v)

    def chunk(j, acc):
        pltpu.async_copy(
            table_hbm.at[idx_v.at[pl.ds(j * C, C)]], rows_v, sem
        ).wait()
        for k in range(C // 16):
            rid = lax.iota(jnp.int32, 16) + (k * 16)
            tv = tgt_v[pl.ds(j * C + k * 16, 16)]
            iv = idx_v[pl.ds(j * C + k * 16, 16)]
            picked = plsc.load_gather(rows_v, [rid, tv])
            ls = plsc.load_gather(lse_v, [iv])
            acc = acc + (ls - picked)
        pltpu.sync_copy(rows_v, out_hbm.at[pl.ds(base + j * C, C)])
        return acc

    acc = lax.fori_loop(0, NCH, chunk, jnp.zeros((16,), jnp.float32))
    tmp_v[...] = acc
    pltpu.sync_copy(tmp_v, part_hbm.at[pl.ds(wid * 32, 16)])
    tmp_v[...] = jnp.zeros((16,), jnp.float32)
    pltpu.sync_copy(tmp_v, part_hbm.at[pl.ds(wid * 32 + 16, 16)])


def kernel(inputs, targets, table):
    idx_flat = inputs.reshape(-1).astype(jnp.int32)
    tgt_flat = targets.reshape(-1).astype(jnp.int32)
    tpad = jnp.pad(table, ((0, VP - V), (0, VP - V)), constant_values=-1e30)
    lse = pl.pallas_call(
        _lse_body,
        out_shape=jax.ShapeDtypeStruct((VP, 1), jnp.float32),
    )(tpad)
    logits_flat, parts = _sc_gather(table, idx_flat, tgt_flat, lse.reshape(VP))
    loss11 = pl.pallas_call(
        _loss_body,
        out_shape=jax.ShapeDtypeStruct((1, 1), jnp.float32),
    )(parts.reshape(8, 128))
    return logits_flat.reshape(1024, 50, V), loss11[0, 0]
